# baseline (speedup 1.0000x reference)
import functools

import jax
import jax.numpy as jnp
from jax import lax
from jax.experimental import pallas as pl
from jax.experimental.pallas import tpu as pltpu
from jax.experimental.pallas import tpu_sc as plsc

_N = 10000
_E = 320000
_D = 128

_NC = 2
_NS = 16
_NW = _NC * _NS

_CHUNK = 128
_GRP = 8
_GRPS = 10
_CHUNKS = _GRP * _GRPS
_EPAD = _NW * _CHUNKS * _CHUNK
_NPAD = 10240
_ROWCHUNKS = _NPAD // _CHUNK
_DUMMY = _N

_CH_C0 = 96
_CH_C1 = 64
_CH_PAIR = _CH_C0 + _CH_C1
_ACC_CHUNKS = 79
_NDUMMY = _ACC_CHUNKS * _CHUNK - _N


def _sc_agg_body(h_hbm, src_hbm, dst_hbm, out_hbm,
                 src_v, dst_v, rows_a, rows_b, acc_sh, sem_a, sem_b):
    c = lax.axis_index("c")
    s = lax.axis_index("s")
    base = s * _CH_PAIR + c * _CH_C0

    zero16 = jnp.zeros((16,), jnp.float32)

    def _zrow(i, _):
        for q in range(_D // 16):
            rows_a[i, pl.ds(q * 16, 16)] = zero16
        return 0

    lax.fori_loop(0, _CHUNK, _zrow, 0)

    per_sub = 5
    for k in range(per_sub):
        chunk = s * per_sub + k

        @pl.when(chunk < _ACC_CHUNKS)
        def _():
            pltpu.sync_copy(rows_a, acc_sh.at[pl.ds(chunk * _CHUNK, _CHUNK)])

    @pl.when(c == 0)
    def _():
        pltpu.sync_copy(src_hbm.at[pl.ds(base, _CH_C0)],
                        src_v.at[pl.ds(0, _CH_C0)])

    @pl.when(c == 1)
    def _():
        pltpu.sync_copy(src_hbm.at[pl.ds(base, _CH_C1)],
                        src_v.at[pl.ds(0, _CH_C1)])

    plsc.subcore_barrier()

    bufs = (rows_a, rows_b)
    sems = (sem_a, sem_b)

    def _edge_loop(ngrp):
        pltpu.sync_copy(dst_hbm.at[pl.ds(base, _GRP)], dst_v)
        pltpu.async_copy(h_hbm.at[src_v.at[0]], rows_a, sem_a)

        def _group(g, _):
            for k in range(_GRP):
                cur, nxt = bufs[k % 2], bufs[1 - k % 2]
                csem, nsem = sems[k % 2], sems[1 - k % 2]
                nc = g * _GRP + k + 1
                if k < _GRP - 1:
                    pltpu.async_copy(h_hbm.at[src_v.at[nc]], nxt, nsem)
                else:
                    @pl.when(g < ngrp - 1)
                    def _():
                        pltpu.async_copy(h_hbm.at[src_v.at[nc]], nxt, nsem)
                pltpu.make_async_copy(h_hbm.at[src_v.at[0]], cur, csem).wait()
                pltpu.sync_copy(cur, acc_sh.at[dst_v.at[k]], add=True)

            @pl.when(g < ngrp - 1)
            def _():
                pltpu.sync_copy(
                    dst_hbm.at[pl.ds(base + (g + 1) * _GRP, _GRP)], dst_v)
            return 0

        lax.fori_loop(0, ngrp, _group, 0)

    @pl.when(c == 0)
    def _():
        _edge_loop(_CH_C0 // _GRP)

    @pl.when(c == 1)
    def _():
        _edge_loop(_CH_C1 // _GRP)

    plsc.subcore_barrier()

    for k in range(per_sub):
        chunk = s * per_sub + k

        @pl.when(chunk < _ACC_CHUNKS)
        def _():
            rows = pl.ds(chunk * _CHUNK, _CHUNK)
            pltpu.sync_copy(acc_sh.at[rows], out_hbm.at[c].at[rows])


_sc_agg = pl.kernel(
    _sc_agg_body,
    out_type=[jax.ShapeDtypeStruct((_NC, _NPAD, _D), jnp.float32)],
    mesh=plsc.VectorSubcoreMesh(core_axis_name="c", subcore_axis_name="s"),
    scratch_types=[
        pltpu.VMEM((_CH_C0, _CHUNK), jnp.int32),
        pltpu.VMEM((_GRP, _CHUNK), jnp.int32),
        pltpu.VMEM((_CHUNK, _D), jnp.float32),
        pltpu.VMEM((_CHUNK, _D), jnp.float32),
        pltpu.VMEM_SHARED((_ACC_CHUNKS * _CHUNK, _D), jnp.float32),
        pltpu.SemaphoreType.DMA,
        pltpu.SemaphoreType.DMA,
    ],
)


def _sc_deg_body(dst_hbm, deg_hbm, dst_v, ones_v, deg_sh):
    c = lax.axis_index("c")
    s = lax.axis_index("s")
    wid = s * _NC + c

    zero16 = jnp.zeros((16,), jnp.float32)
    one16 = jnp.ones((16,), jnp.float32)

    def _fill(i, _):
        for q in range(_D // 16):
            ones_v[i, pl.ds(q * 16, 16)] = zero16
        return 0

    lax.fori_loop(0, _CHUNK, _fill, 0)

    per_sub = _ROWCHUNKS // _NS
    for k in range(per_sub):
        chunk = s * per_sub + k
        pltpu.sync_copy(ones_v, deg_sh.at[pl.ds(chunk * _CHUNK, _CHUNK)])

    def _fill1(i, _):
        for q in range(_D // 16):
            ones_v[i, pl.ds(q * 16, 16)] = one16
        return 0

    lax.fori_loop(0, _CHUNK, _fill1, 0)

    plsc.subcore_barrier()

    def _group(g, _):
        base = wid * _CHUNKS + g * _GRP
        pltpu.sync_copy(dst_hbm.at[pl.ds(base, _GRP)], dst_v)
        for j in range(_GRP):
            pltpu.sync_copy(ones_v, deg_sh.at[dst_v.at[j]], add=True)
        return 0

    lax.fori_loop(0, _GRPS, _group, 0)

    plsc.subcore_barrier()

    for k in range(per_sub):
        chunk = s * per_sub + k
        rows = pl.ds(chunk * _CHUNK, _CHUNK)
        pltpu.sync_copy(deg_sh.at[rows], deg_hbm.at[c].at[rows])


_sc_deg = pl.kernel(
    _sc_deg_body,
    out_type=[jax.ShapeDtypeStruct((_NC, _NPAD, _D), jnp.float32)],
    mesh=plsc.VectorSubcoreMesh(core_axis_name="c", subcore_axis_name="s"),
    scratch_types=[
        pltpu.VMEM((_GRP, _CHUNK), jnp.int32),
        pltpu.VMEM((_CHUNK, _D), jnp.float32),
        pltpu.VMEM_SHARED((_NPAD, _D), jnp.float32),
    ],
)


_BLK = 1024


def _map_body(x_ref, w_ref, b_ref, o_ref):
    o_ref[...] = (
        jnp.dot(x_ref[...], w_ref[...], preferred_element_type=jnp.float32)
        + b_ref[...]
    )


def _tc_map(x, w, b):
    return pl.pallas_call(
        _map_body,
        grid=(_NPAD // _BLK,),
        in_specs=[
            pl.BlockSpec((_BLK, _D), lambda i: (i, 0)),
            pl.BlockSpec((_D, _D), lambda i: (0, 0)),
            pl.BlockSpec((1, _D), lambda i: (0, 0)),
        ],
        out_specs=pl.BlockSpec((_BLK, _D), lambda i: (i, 0)),
        out_shape=jax.ShapeDtypeStruct((_NPAD, _D), jnp.float32),
    )(x, w, b.reshape(1, _D))


def _scale_body(deg_ref, s_ref):
    deg = deg_ref[0][:, :1] + deg_ref[1][:, :1]
    s_ref[...] = 1.0 / (deg + 1.0)


def _tc_scale(deg):
    return pl.pallas_call(
        _scale_body,
        grid=(_NPAD // _BLK,),
        in_specs=[pl.BlockSpec((_NC, _BLK, _D), lambda i: (0, i, 0))],
        out_specs=pl.BlockSpec((_BLK, 1), lambda i: (i, 0)),
        out_shape=jax.ShapeDtypeStruct((_NPAD, 1), jnp.float32),
    )(deg)


def _layer_body(relu, agg_ref, s_ref, h_ref, w_ref, b_ref, o_ref):
    hn = (agg_ref[0] + agg_ref[1] + h_ref[...]) * s_ref[...]
    out = (
        jnp.dot(hn, w_ref[...], preferred_element_type=jnp.float32)
        + b_ref[...]
    )
    if relu:
        out = jnp.maximum(out, 0.0)
    o_ref[...] = out


def _tc_layer(agg, scale, h, w, b, relu):
    return pl.pallas_call(
        functools.partial(_layer_body, relu),
        grid=(_NPAD // _BLK,),
        in_specs=[
            pl.BlockSpec((_NC, _BLK, _D), lambda i: (0, i, 0)),
            pl.BlockSpec((_BLK, 1), lambda i: (i, 0)),
            pl.BlockSpec((_BLK, _D), lambda i: (i, 0)),
            pl.BlockSpec((_D, _D), lambda i: (0, 0)),
            pl.BlockSpec((1, _D), lambda i: (0, 0)),
        ],
        out_specs=pl.BlockSpec((_BLK, _D), lambda i: (i, 0)),
        out_shape=jax.ShapeDtypeStruct((_NPAD, _D), jnp.float32),
    )(agg, scale, h, w, b.reshape(1, _D))


def kernel(x, edge_index, W_map, b_map, W0, b0, W1, b1, W2, b2):
    src = edge_index[0]
    dst = edge_index[1]

    pad_e = _EPAD - _E
    pad_dst = _DUMMY + jnp.arange(pad_e, dtype=jnp.int32) % _NDUMMY
    src_p = jnp.concatenate([src, jnp.zeros((pad_e,), jnp.int32)])
    dst_p = jnp.concatenate([dst, pad_dst])
    src_p = src_p.reshape(_EPAD // _CHUNK, _CHUNK)
    dst_p = dst_p.reshape(_EPAD // _CHUNK, _CHUNK)

    x_p = jnp.zeros((_NPAD, _D), jnp.float32).at[:_N].set(x)

    (deg,) = _sc_deg(dst_p)
    scale = _tc_scale(deg)
    h0 = _tc_map(x_p, W_map, b_map)
    (agg,) = _sc_agg(h0, src_p, dst_p)
    h1 = _tc_layer(agg, scale, h0, W0, b0, True)
    (agg,) = _sc_agg(h1, src_p, dst_p)
    h2 = _tc_layer(agg, scale, h1, W1, b1, True)
    (agg,) = _sc_agg(h2, src_p, dst_p)
    h3 = _tc_layer(agg, scale, h2, W2, b2, False)
    return h3[:_N]


# speedup vs baseline: 1.1396x; 1.1396x over previous
import functools

import jax
import jax.numpy as jnp
from jax import lax
from jax.experimental import pallas as pl
from jax.experimental.pallas import tpu as pltpu
from jax.experimental.pallas import tpu_sc as plsc

_N = 10000
_E = 320000
_D = 128

_NC = 2
_NS = 16
_NW = _NC * _NS

_CHUNK = 128
_GRP = 8
_GRPS = 10
_CHUNKS = _GRP * _GRPS
_EPAD = _NW * _CHUNKS * _CHUNK
_NPAD = 10240
_ROWCHUNKS = _NPAD // _CHUNK
_DUMMY = _N

_CH_C0 = 120
_CH_C1 = 40
_CH_PAIR = _CH_C0 + _CH_C1
_ACC_CHUNKS = 79
_NDUMMY = _ACC_CHUNKS * _CHUNK - _N


def _sc_agg_body(h_hbm, src_hbm, dst_hbm, out_hbm,
                 src_v, dst_v, rows_a, rows_b, acc_sh, sem_a, sem_b):
    c = lax.axis_index("c")
    s = lax.axis_index("s")
    base = s * _CH_PAIR + c * _CH_C0

    zero16 = jnp.zeros((16,), jnp.float32)

    def _zrow(i, _):
        for q in range(_D // 16):
            rows_a[i, pl.ds(q * 16, 16)] = zero16
        return 0

    lax.fori_loop(0, _CHUNK, _zrow, 0)

    per_sub = 5
    for k in range(per_sub):
        chunk = s * per_sub + k

        @pl.when(chunk < _ACC_CHUNKS)
        def _():
            pltpu.sync_copy(rows_a, acc_sh.at[pl.ds(chunk * _CHUNK, _CHUNK)])

    @pl.when(c == 0)
    def _():
        pltpu.sync_copy(src_hbm.at[pl.ds(base, _CH_C0)],
                        src_v.at[pl.ds(0, _CH_C0)])

    @pl.when(c == 1)
    def _():
        pltpu.sync_copy(src_hbm.at[pl.ds(base, _CH_C1)],
                        src_v.at[pl.ds(0, _CH_C1)])

    plsc.subcore_barrier()

    bufs = (rows_a, rows_b)
    sems = (sem_a, sem_b)

    def _edge_loop(ngrp):
        pltpu.sync_copy(dst_hbm.at[pl.ds(base, _GRP)], dst_v)
        pltpu.async_copy(h_hbm.at[src_v.at[0]], rows_a, sem_a)

        def _group(g, _):
            for k in range(_GRP):
                cur, nxt = bufs[k % 2], bufs[1 - k % 2]
                csem, nsem = sems[k % 2], sems[1 - k % 2]
                nc = g * _GRP + k + 1
                if k < _GRP - 1:
                    pltpu.async_copy(h_hbm.at[src_v.at[nc]], nxt, nsem)
                else:
                    @pl.when(g < ngrp - 1)
                    def _():
                        pltpu.async_copy(h_hbm.at[src_v.at[nc]], nxt, nsem)
                pltpu.make_async_copy(h_hbm.at[src_v.at[0]], cur, csem).wait()
                pltpu.sync_copy(cur, acc_sh.at[dst_v.at[k]], add=True)

            @pl.when(g < ngrp - 1)
            def _():
                pltpu.sync_copy(
                    dst_hbm.at[pl.ds(base + (g + 1) * _GRP, _GRP)], dst_v)
            return 0

        lax.fori_loop(0, ngrp, _group, 0)

    @pl.when(c == 0)
    def _():
        _edge_loop(_CH_C0 // _GRP)

    @pl.when(c == 1)
    def _():
        _edge_loop(_CH_C1 // _GRP)

    plsc.subcore_barrier()

    for k in range(per_sub):
        chunk = s * per_sub + k

        @pl.when(chunk < _ACC_CHUNKS)
        def _():
            rows = pl.ds(chunk * _CHUNK, _CHUNK)
            pltpu.sync_copy(acc_sh.at[rows], out_hbm.at[c].at[rows])


_sc_agg = pl.kernel(
    _sc_agg_body,
    out_type=[jax.ShapeDtypeStruct((_NC, _NPAD, _D), jnp.float32)],
    mesh=plsc.VectorSubcoreMesh(core_axis_name="c", subcore_axis_name="s"),
    scratch_types=[
        pltpu.VMEM((_CH_C0, _CHUNK), jnp.int32),
        pltpu.VMEM((_GRP, _CHUNK), jnp.int32),
        pltpu.VMEM((_CHUNK, _D), jnp.float32),
        pltpu.VMEM((_CHUNK, _D), jnp.float32),
        pltpu.VMEM_SHARED((_ACC_CHUNKS * _CHUNK, _D), jnp.float32),
        pltpu.SemaphoreType.DMA,
        pltpu.SemaphoreType.DMA,
    ],
)


def _sc_deg_body(dst_hbm, deg_hbm, dst_v, ones_v, deg_sh):
    c = lax.axis_index("c")
    s = lax.axis_index("s")
    wid = s * _NC + c

    zero16 = jnp.zeros((16,), jnp.float32)
    one16 = jnp.ones((16,), jnp.float32)

    def _fill(i, _):
        for q in range(_D // 16):
            ones_v[i, pl.ds(q * 16, 16)] = zero16
        return 0

    lax.fori_loop(0, _CHUNK, _fill, 0)

    per_sub = _ROWCHUNKS // _NS
    for k in range(per_sub):
        chunk = s * per_sub + k
        pltpu.sync_copy(ones_v, deg_sh.at[pl.ds(chunk * _CHUNK, _CHUNK)])

    def _fill1(i, _):
        for q in range(_D // 16):
            ones_v[i, pl.ds(q * 16, 16)] = one16
        return 0

    lax.fori_loop(0, _CHUNK, _fill1, 0)

    plsc.subcore_barrier()

    def _group(g, _):
        base = wid * _CHUNKS + g * _GRP
        pltpu.sync_copy(dst_hbm.at[pl.ds(base, _GRP)], dst_v)
        for j in range(_GRP):
            pltpu.sync_copy(ones_v, deg_sh.at[dst_v.at[j]], add=True)
        return 0

    lax.fori_loop(0, _GRPS, _group, 0)

    plsc.subcore_barrier()

    for k in range(per_sub):
        chunk = s * per_sub + k
        rows = pl.ds(chunk * _CHUNK, _CHUNK)
        pltpu.sync_copy(deg_sh.at[rows], deg_hbm.at[c].at[rows])


_sc_deg = pl.kernel(
    _sc_deg_body,
    out_type=[jax.ShapeDtypeStruct((_NC, _NPAD, _D), jnp.float32)],
    mesh=plsc.VectorSubcoreMesh(core_axis_name="c", subcore_axis_name="s"),
    scratch_types=[
        pltpu.VMEM((_GRP, _CHUNK), jnp.int32),
        pltpu.VMEM((_CHUNK, _D), jnp.float32),
        pltpu.VMEM_SHARED((_NPAD, _D), jnp.float32),
    ],
)


_BLK = 1024


def _map_body(x_ref, w_ref, b_ref, o_ref):
    o_ref[...] = (
        jnp.dot(x_ref[...], w_ref[...], preferred_element_type=jnp.float32)
        + b_ref[...]
    )


def _tc_map(x, w, b):
    return pl.pallas_call(
        _map_body,
        grid=(_NPAD // _BLK,),
        in_specs=[
            pl.BlockSpec((_BLK, _D), lambda i: (i, 0)),
            pl.BlockSpec((_D, _D), lambda i: (0, 0)),
            pl.BlockSpec((1, _D), lambda i: (0, 0)),
        ],
        out_specs=pl.BlockSpec((_BLK, _D), lambda i: (i, 0)),
        out_shape=jax.ShapeDtypeStruct((_NPAD, _D), jnp.float32),
    )(x, w, b.reshape(1, _D))


def _scale_body(deg_ref, s_ref):
    deg = deg_ref[0][:, :1] + deg_ref[1][:, :1]
    s_ref[...] = 1.0 / (deg + 1.0)


def _tc_scale(deg):
    return pl.pallas_call(
        _scale_body,
        grid=(_NPAD // _BLK,),
        in_specs=[pl.BlockSpec((_NC, _BLK, _D), lambda i: (0, i, 0))],
        out_specs=pl.BlockSpec((_BLK, 1), lambda i: (i, 0)),
        out_shape=jax.ShapeDtypeStruct((_NPAD, 1), jnp.float32),
    )(deg)


def _layer_body(relu, agg_ref, s_ref, h_ref, w_ref, b_ref, o_ref):
    hn = (agg_ref[0] + agg_ref[1] + h_ref[...]) * s_ref[...]
    out = (
        jnp.dot(hn, w_ref[...], preferred_element_type=jnp.float32)
        + b_ref[...]
    )
    if relu:
        out = jnp.maximum(out, 0.0)
    o_ref[...] = out


def _tc_layer(agg, scale, h, w, b, relu):
    return pl.pallas_call(
        functools.partial(_layer_body, relu),
        grid=(_NPAD // _BLK,),
        in_specs=[
            pl.BlockSpec((_NC, _BLK, _D), lambda i: (0, i, 0)),
            pl.BlockSpec((_BLK, 1), lambda i: (i, 0)),
            pl.BlockSpec((_BLK, _D), lambda i: (i, 0)),
            pl.BlockSpec((_D, _D), lambda i: (0, 0)),
            pl.BlockSpec((1, _D), lambda i: (0, 0)),
        ],
        out_specs=pl.BlockSpec((_BLK, _D), lambda i: (i, 0)),
        out_shape=jax.ShapeDtypeStruct((_NPAD, _D), jnp.float32),
    )(agg, scale, h, w, b.reshape(1, _D))


def kernel(x, edge_index, W_map, b_map, W0, b0, W1, b1, W2, b2):
    src = edge_index[0]
    dst = edge_index[1]

    pad_e = _EPAD - _E
    pad_dst = _DUMMY + jnp.arange(pad_e, dtype=jnp.int32) % _NDUMMY
    src_p = jnp.concatenate([src, jnp.zeros((pad_e,), jnp.int32)])
    dst_p = jnp.concatenate([dst, pad_dst])
    src_p = src_p.reshape(_EPAD // _CHUNK, _CHUNK)
    dst_p = dst_p.reshape(_EPAD // _CHUNK, _CHUNK)

    x_p = jnp.zeros((_NPAD, _D), jnp.float32).at[:_N].set(x)

    (deg,) = _sc_deg(dst_p)
    scale = _tc_scale(deg)
    h0 = _tc_map(x_p, W_map, b_map)
    (agg,) = _sc_agg(h0, src_p, dst_p)
    h1 = _tc_layer(agg, scale, h0, W0, b0, True)
    (agg,) = _sc_agg(h1, src_p, dst_p)
    h2 = _tc_layer(agg, scale, h1, W1, b1, True)
    (agg,) = _sc_agg(h2, src_p, dst_p)
    h3 = _tc_layer(agg, scale, h2, W2, b2, False)
    return h3[:_N]
